# baseline (speedup 1.0000x reference)
import jax
import jax.numpy as jnp
from jax.experimental import pallas as pl
from jax.experimental.pallas import tpu as pltpu

H0, W0 = 64, 64
C1, C2, C3 = 32, 64, 128
FEAT = C3 * 8 * 8
HID = 256
NOUT = 2
TB = 8
MB = 256
F32 = jnp.float32
BF16 = jnp.bfloat16


def _tower_kernel(x_ref, a1h_ref, a1l_ref, a1m_ref, bb1_ref, w2_ref, b2_ref,
                  w3_ref, b3_ref,
                  f_ref, xh_ref, xl_ref, xm_ref, p2_ref, p3_ref):
    tb = x_ref.shape[1]
    zrow = jnp.zeros((tb, 64), BF16)

    def band3(p_ref, a_ref):
        acc = jnp.dot(p_ref[0:64 * tb, :], a_ref[0:64, :],
                      preferred_element_type=F32)
        acc += jnp.dot(p_ref[tb:65 * tb, :], a_ref[64:128, :],
                       preferred_element_type=F32)
        acc += jnp.dot(p_ref[2 * tb:66 * tb, :], a_ref[128:192, :],
                       preferred_element_type=F32)
        return acc

    for r in (xh_ref, xl_ref, xm_ref):
        r[0:tb, :] = zrow
        r[65 * tb:66 * tb, :] = zrow
    xv = x_ref[...].reshape(64 * tb, 64)
    xh = xv.astype(BF16)
    xr = xv - xh.astype(F32)
    xl = xr.astype(BF16)
    xh_ref[tb:65 * tb, :] = xh
    xl_ref[tb:65 * tb, :] = xl
    xm_ref[tb:65 * tb, :] = (xr - xl.astype(F32)).astype(BF16)
    acc1 = band3(xh_ref, a1h_ref)
    acc1 += band3(xl_ref, a1h_ref)
    acc1 += band3(xh_ref, a1l_ref)
    acc1 += band3(xl_ref, a1l_ref)
    acc1 += band3(xm_ref, a1h_ref)
    acc1 += band3(xh_ref, a1m_ref)
    wm = jnp.maximum(acc1[:, :1024], acc1[:, 1024:])
    v = wm.reshape(32, 2 * tb, 1024)
    hm = jnp.maximum(v[:, :tb, :], v[:, tb:, :])
    hm = hm.reshape(32 * tb, 1024)
    y1 = jnp.maximum(hm + bb1_ref[...], 0.0).astype(BF16)

    p2_ref[0:1, :, :] = jnp.zeros((1, 34 * tb, C1), BF16)
    p2_ref[33:34, :, :] = jnp.zeros((1, 34 * tb, C1), BF16)
    p2_ref[:, 0:tb, :] = jnp.zeros((34, tb, C1), BF16)
    p2_ref[:, 33 * tb:34 * tb, :] = jnp.zeros((34, tb, C1), BF16)
    for w in range(32):
        p2_ref[1:33, (w + 1) * tb:(w + 2) * tb, :] = (
            y1[:, w * C1:(w + 1) * C1].reshape(32, tb, C1))

    cols = [p2_ref[dy:dy + 32, dx * tb:(32 + dx) * tb, :]
            .reshape(32 * 32 * tb, C1) for dy in range(3) for dx in range(3)]
    patch2 = jnp.concatenate(cols, axis=1)
    acc2 = jnp.dot(patch2, w2_ref[...], preferred_element_type=F32)
    y2 = jnp.maximum(acc2 + b2_ref[...], 0.0)
    y2 = y2.reshape(32, 16, 2, tb, C2).max(axis=2)
    y2 = y2.reshape(16, 2, 16 * tb, C2).max(axis=1)
    y2 = y2.reshape(16, 16 * tb, C2).astype(BF16)

    p3_ref[0:1, :, :] = jnp.zeros((1, 18 * tb, C2), BF16)
    p3_ref[17:18, :, :] = jnp.zeros((1, 18 * tb, C2), BF16)
    p3_ref[:, 0:tb, :] = jnp.zeros((18, tb, C2), BF16)
    p3_ref[:, 17 * tb:18 * tb, :] = jnp.zeros((18, tb, C2), BF16)
    p3_ref[1:17, tb:17 * tb, :] = y2

    cols = [p3_ref[dy:dy + 16, dx * tb:(16 + dx) * tb, :]
            .reshape(16 * 16 * tb, C2) for dy in range(3) for dx in range(3)]
    patch3 = jnp.concatenate(cols, axis=1)
    acc3 = jnp.dot(patch3, w3_ref[...], preferred_element_type=F32)
    y3 = jnp.maximum(acc3 + b3_ref[...], 0.0)
    y3 = y3.reshape(16, 8, 2, tb, C3).max(axis=2)
    y3 = y3.reshape(8, 2, 8 * tb, C3).max(axis=1)
    y3 = y3.reshape(64 * tb, C3).astype(BF16)

    for j in range(64):
        f_ref[:, j * C3:(j + 1) * C3] = y3[j * tb:(j + 1) * tb, :]


def _mlp_kernel(f_ref, wl1_ref, bl1_ref, wl2_ref, bl2_ref, wl3_ref, bl3_ref,
                o_ref):
    h = jnp.dot(f_ref[...], wl1_ref[...], preferred_element_type=F32)
    h = jnp.maximum(h + bl1_ref[...], 0.0).astype(BF16)
    h = jnp.dot(h, wl2_ref[...], preferred_element_type=F32)
    h = jnp.maximum(h + bl2_ref[...], 0.0).astype(BF16)
    o_ref[...] = (jnp.dot(h, wl3_ref[...], preferred_element_type=F32)
                  + bl3_ref[...])


def _banded_mat(wmat, n, cin, cout):
    dt = wmat.dtype
    w9 = wmat.reshape(3, 3, cin, cout)
    iw = jnp.arange(n)
    wcol = jnp.concatenate([2 * jnp.arange(n // 2), 2 * jnp.arange(n // 2) + 1])
    masks = jnp.stack([(iw[:, None] == wcol[None, :] + dx - 1)
                       .astype(dt) for dx in range(3)])
    a = jnp.einsum('xuw,dxio->duiwo', masks, w9)
    return a.reshape(3 * n * cin, n * cout)


def kernel(x_nchw, w1, b1, w2, b2, w3, b3, wl1, bl1, wl2, bl2, wl3, bl3):
    B = x_nchw.shape[0]
    x = x_nchw.reshape(B, H0, W0)
    Bp = ((B + MB - 1) // MB) * MB
    if Bp != B:
        x = jnp.concatenate([x, jnp.zeros((Bp - B, H0, W0), x.dtype)], axis=0)
    xt = jnp.transpose(x, (1, 0, 2))

    a1 = _banded_mat(w1, 64, 1, C1)
    a1h = a1.astype(BF16)
    a1r = a1 - a1h.astype(F32)
    a1l = a1r.astype(BF16)
    a1m = (a1r - a1l.astype(F32)).astype(BF16)
    bb1 = jnp.tile(b1.reshape(-1), 32).reshape(1, 32 * C1)

    const2 = lambda b: (0, 0)
    feat = pl.pallas_call(
        _tower_kernel,
        out_shape=jax.ShapeDtypeStruct((Bp, FEAT), BF16),
        grid=(Bp // TB,),
        in_specs=[
            pl.BlockSpec((H0, TB, W0), lambda b: (0, b, 0)),
            pl.BlockSpec((192, 2048), const2),
            pl.BlockSpec((192, 2048), const2),
            pl.BlockSpec((192, 2048), const2),
            pl.BlockSpec((1, 32 * C1), const2),
            pl.BlockSpec((9 * C1, C2), const2), pl.BlockSpec((1, C2), const2),
            pl.BlockSpec((9 * C2, C3), const2), pl.BlockSpec((1, C3), const2),
        ],
        out_specs=pl.BlockSpec((TB, FEAT), lambda b: (b, 0)),
        scratch_shapes=[
            pltpu.VMEM((66 * TB, 64), BF16),
            pltpu.VMEM((66 * TB, 64), BF16),
            pltpu.VMEM((66 * TB, 64), BF16),
            pltpu.VMEM((34, 34 * TB, C1), BF16),
            pltpu.VMEM((18, 18 * TB, C2), BF16),
        ],
        compiler_params=pltpu.CompilerParams(
            dimension_semantics=("parallel",),
            vmem_limit_bytes=48 * 1024 * 1024,
        ),
    )(xt, a1h, a1l, a1m, bb1, w2, b2, w3, b3)

    out = pl.pallas_call(
        _mlp_kernel,
        out_shape=jax.ShapeDtypeStruct((Bp, NOUT), jnp.float32),
        grid=(Bp // MB,),
        in_specs=[
            pl.BlockSpec((MB, FEAT), lambda b: (b, 0)),
            pl.BlockSpec((FEAT, HID), const2), pl.BlockSpec((1, HID), const2),
            pl.BlockSpec((HID, HID), const2),  pl.BlockSpec((1, HID), const2),
            pl.BlockSpec((HID, NOUT), const2), pl.BlockSpec((1, NOUT), const2),
        ],
        out_specs=pl.BlockSpec((MB, NOUT), lambda b: (b, 0)),
        compiler_params=pltpu.CompilerParams(
            dimension_semantics=("parallel",),
            vmem_limit_bytes=48 * 1024 * 1024,
        ),
    )(feat, wl1, bl1, wl2, bl2, wl3, bl3)
    return out[:B]


# speedup vs baseline: 1.4892x; 1.4892x over previous
import jax
import jax.numpy as jnp
from jax.experimental import pallas as pl
from jax.experimental.pallas import tpu as pltpu

H0, W0 = 64, 64
C1, C2, C3 = 32, 64, 128
FEAT = C3 * 8 * 8
HID = 256
NOUT = 2
TB = 8
MB = 256
F32 = jnp.float32


def _tower_kernel(x_ref, a1h_ref, a1l_ref, bb1_ref, a2_ref, bb2_ref,
                  a3_ref, bb3_ref,
                  f_ref, xh_ref, xl_ref, y1p_ref, y2p_ref):
    tb = x_ref.shape[1]
    zrow = jnp.zeros((tb, 64), jnp.bfloat16)
    zlane = jnp.zeros((tb, 1024), jnp.bfloat16)

    def band3(p_ref, a_ref, rows, chunk):
        acc = jnp.dot(p_ref[0:rows * tb, :], a_ref[0:chunk, :],
                      preferred_element_type=F32)
        acc += jnp.dot(p_ref[tb:(rows + 1) * tb, :],
                       a_ref[chunk:2 * chunk, :], preferred_element_type=F32)
        acc += jnp.dot(p_ref[2 * tb:(rows + 2) * tb, :],
                       a_ref[2 * chunk:3 * chunk, :],
                       preferred_element_type=F32)
        return acc

    def pool_bias_relu(acc, bb_ref, half_rows):
        half = acc.shape[-1] // 2
        wm = jnp.maximum(acc[:, :half], acc[:, half:])
        v = wm.reshape(half_rows, 2 * tb, half)
        hm = jnp.maximum(v[:, :tb, :], v[:, tb:, :])
        hm = hm.reshape(half_rows * tb, half)
        return jnp.maximum(hm + bb_ref[...], 0.0).astype(jnp.bfloat16)

    xh_ref[0:tb, :] = zrow
    xh_ref[65 * tb:66 * tb, :] = zrow
    xl_ref[0:tb, :] = zrow
    xl_ref[65 * tb:66 * tb, :] = zrow
    xv = x_ref[...].reshape(64 * tb, 64)
    xh = xv.astype(jnp.bfloat16)
    xh_ref[tb:65 * tb, :] = xh
    xl_ref[tb:65 * tb, :] = (xv - xh.astype(F32)).astype(jnp.bfloat16)
    acc1 = band3(xh_ref, a1h_ref, 64, 64)
    acc1 += band3(xl_ref, a1h_ref, 64, 64)
    acc1 += band3(xh_ref, a1l_ref, 64, 64)
    y1 = pool_bias_relu(acc1, bb1_ref, 32)

    y1p_ref[0:tb, :] = zlane
    y1p_ref[33 * tb:34 * tb, :] = zlane
    y1p_ref[tb:33 * tb, :] = y1
    acc2 = band3(y1p_ref, a2_ref, 32, 1024)
    y2 = pool_bias_relu(acc2, bb2_ref, 16)

    y2p_ref[0:tb, :] = zlane
    y2p_ref[17 * tb:18 * tb, :] = zlane
    y2p_ref[tb:17 * tb, :] = y2
    acc3 = band3(y2p_ref, a3_ref, 16, 1024)
    y3 = pool_bias_relu(acc3, bb3_ref, 8)

    f_ref[...] = y3.reshape(8, tb, 1024)


def _mlp_kernel(f_ref, wl1_ref, bl1_ref, wl2_ref, bl2_ref, wl3_ref, bl3_ref,
                o_ref):
    h = jnp.dot(f_ref[0], wl1_ref[0:1024, :], preferred_element_type=F32)
    for j in range(1, 8):
        h += jnp.dot(f_ref[j], wl1_ref[j * 1024:(j + 1) * 1024, :],
                     preferred_element_type=F32)
    h = jnp.maximum(h + bl1_ref[...], 0.0).astype(jnp.bfloat16)
    h = jnp.dot(h, wl2_ref[...], preferred_element_type=F32)
    h = jnp.maximum(h + bl2_ref[...], 0.0).astype(jnp.bfloat16)
    o_ref[...] = (jnp.dot(h, wl3_ref[...], preferred_element_type=F32)
                  + bl3_ref[...])


def _banded_mat(wmat, n, cin, cout):
    dt = wmat.dtype
    w9 = wmat.reshape(3, 3, cin, cout)
    iw = jnp.arange(n)
    wcol = jnp.concatenate([2 * jnp.arange(n // 2), 2 * jnp.arange(n // 2) + 1])
    masks = jnp.stack([(iw[:, None] == wcol[None, :] + dx - 1)
                       .astype(dt) for dx in range(3)])
    a = jnp.einsum('xuw,dxio->duiwo', masks, w9)
    return a.reshape(3 * n * cin, n * cout)


def kernel(x_nchw, w1, b1, w2, b2, w3, b3, wl1, bl1, wl2, bl2, wl3, bl3):
    B = x_nchw.shape[0]
    x = x_nchw.reshape(B, H0, W0)
    Bp = ((B + MB - 1) // MB) * MB
    if Bp != B:
        x = jnp.concatenate([x, jnp.zeros((Bp - B, H0, W0), x.dtype)], axis=0)
    xt = jnp.transpose(x, (1, 0, 2))

    a1 = _banded_mat(w1, 64, 1, C1)
    a1h = a1.astype(jnp.bfloat16)
    a1l = (a1 - a1h.astype(jnp.float32)).astype(jnp.bfloat16)
    a2 = _banded_mat(w2, 32, C1, C2)
    a3 = _banded_mat(w3, 16, C2, C3)
    bb1 = jnp.tile(b1.reshape(-1), 32).reshape(1, 32 * C1)
    bb2 = jnp.tile(b2.reshape(-1), 16).reshape(1, 16 * C2)
    bb3 = jnp.tile(b3.reshape(-1), 8).reshape(1, 8 * C3)

    const2 = lambda b: (0, 0)
    feat = pl.pallas_call(
        _tower_kernel,
        out_shape=jax.ShapeDtypeStruct((8, Bp, 1024), jnp.bfloat16),
        grid=(Bp // TB,),
        in_specs=[
            pl.BlockSpec((H0, TB, W0), lambda b: (0, b, 0)),
            pl.BlockSpec((192, 2048), const2),
            pl.BlockSpec((192, 2048), const2),
            pl.BlockSpec((1, 32 * C1), const2),
            pl.BlockSpec((3072, 2048), const2),
            pl.BlockSpec((1, 16 * C2), const2),
            pl.BlockSpec((3072, 2048), const2),
            pl.BlockSpec((1, 8 * C3), const2),
        ],
        out_specs=pl.BlockSpec((8, TB, 1024), lambda b: (0, b, 0)),
        scratch_shapes=[
            pltpu.VMEM((66 * TB, 64), jnp.bfloat16),
            pltpu.VMEM((66 * TB, 64), jnp.bfloat16),
            pltpu.VMEM((34 * TB, 1024), jnp.bfloat16),
            pltpu.VMEM((18 * TB, 1024), jnp.bfloat16),
        ],
        compiler_params=pltpu.CompilerParams(
            dimension_semantics=("parallel",),
            vmem_limit_bytes=60 * 1024 * 1024,
        ),
    )(xt, a1h, a1l, bb1, a2, bb2, a3, bb3)

    out = pl.pallas_call(
        _mlp_kernel,
        out_shape=jax.ShapeDtypeStruct((Bp, NOUT), jnp.float32),
        grid=(Bp // MB,),
        in_specs=[
            pl.BlockSpec((8, MB, 1024), lambda b: (0, b, 0)),
            pl.BlockSpec((FEAT, HID), const2), pl.BlockSpec((1, HID), const2),
            pl.BlockSpec((HID, HID), const2),  pl.BlockSpec((1, HID), const2),
            pl.BlockSpec((HID, NOUT), const2), pl.BlockSpec((1, NOUT), const2),
        ],
        out_specs=pl.BlockSpec((MB, NOUT), lambda b: (b, 0)),
        compiler_params=pltpu.CompilerParams(
            dimension_semantics=("parallel",),
            vmem_limit_bytes=48 * 1024 * 1024,
        ),
    )(feat, wl1, bl1, wl2, bl2, wl3, bl3)
    return out[:B]
